# Initial kernel scaffold; baseline (speedup 1.0000x reference)
import jax
import jax.numpy as jnp
from jax.experimental import pallas as pl


def kernel(cls_scores, bbox_deltas):
    raise NotImplementedError("write your pallas kernel here")


# speedup vs baseline: 10.5945x; 10.5945x over previous
import jax
import jax.numpy as jnp
from jax import lax
from jax.experimental import pallas as pl

_RATIOS = (0.5, 1.0, 2.0)
_SCALES = (128.0, 256.0, 512.0)
_IMAGE_SIZE = 1920
_NMS_PRE = 3000
_NMS_POST = 300
_THRESHOLD = 0.6

_NPAD = 3072
_ROWS = 24
_LANES = 128


def _anchors(feat_stride, map_size):
    xs = (feat_stride * jnp.arange(map_size, dtype=jnp.float64) + feat_stride) / 2.0
    ys = xs
    combos = [(r, s) for r in _RATIOS for s in _SCALES]
    r = jnp.array([c[0] for c in combos], dtype=jnp.float64)
    s = jnp.array([c[1] for c in combos], dtype=jnp.float64)
    X = xs[:, None, None]
    Y = ys[None, :, None]
    x1 = X - s / 2.0
    x2 = X + s / 2.0
    y1 = Y - s * r / 2.0
    y2 = Y + s * r / 2.0
    x1, y1, x2, y2 = jnp.broadcast_arrays(x1, y1, x2, y2)
    return jnp.stack([x1, y1, x2, y2], axis=-1)


def _nms_kernel(anc_ref, dlt_ref, mask_ref, boxes_ref):
    x1 = jnp.clip(anc_ref[0] + dlt_ref[0], 0.0, float(_IMAGE_SIZE))
    y1 = jnp.clip(anc_ref[1] + dlt_ref[1], 0.0, float(_IMAGE_SIZE))
    x2 = jnp.clip(anc_ref[2] + dlt_ref[2], 0.0, float(_IMAGE_SIZE))
    y2 = jnp.clip(anc_ref[3] + dlt_ref[3], 0.0, float(_IMAGE_SIZE))
    boxes_ref[0] = x1
    boxes_ref[1] = y1
    boxes_ref[2] = x2
    boxes_ref[3] = y2

    area = (x2 - x1) * (y2 - y1)
    gidx = (
        lax.broadcasted_iota(jnp.int32, (_ROWS, _LANES), 0) * _LANES
        + lax.broadcasted_iota(jnp.int32, (_ROWS, _LANES), 1)
    )

    def step(i, mask):
        sel = gidx == i

        def pick(v):
            return jnp.sum(jnp.where(sel, v, 0.0))

        xi1 = pick(x1)
        yi1 = pick(y1)
        xi2 = pick(x2)
        yi2 = pick(y2)
        mi = jnp.sum(jnp.where(sel, mask, 0.0))
        ai = (xi2 - xi1) * (yi2 - yi1)

        xx1 = jnp.maximum(xi1, x1)
        yy1 = jnp.maximum(yi1, y1)
        xx2 = jnp.minimum(xi2, x2)
        yy2 = jnp.minimum(yi2, y2)
        w = jnp.clip(xx2 - xx1, 0.0, None)
        h = jnp.clip(yy2 - yy1, 0.0, None)
        inter = w * h
        iou = inter / (ai + area - inter + 1e-9)
        sup = (iou > _THRESHOLD) & (gidx > i) & (mi > 0.5)
        return jnp.where(sup, 0.0, mask)

    mask = lax.fori_loop(0, _NMS_PRE, step, jnp.ones((_ROWS, _LANES), jnp.float32))
    mask_ref[...] = mask


def kernel(cls_scores, bbox_deltas):
    H = cls_scores.shape[2]
    W = cls_scores.shape[3]
    batch_size = cls_scores.shape[0]
    feat_stride = round(_IMAGE_SIZE / float(W))
    anc = _anchors(feat_stride, W).astype(jnp.float32)

    scores_flat = cls_scores[0, ::2].reshape(batch_size, -1)

    bd = jnp.transpose(bbox_deltas, (0, 2, 3, 1))[0]
    bd = bd.reshape(H, W, -1, 4)

    def scramble(a):
        a = a.reshape(batch_size, -1, 4, H, W)
        a = jnp.transpose(a, (0, 3, 4, 1, 2)).reshape(batch_size, -1, 4)
        return a

    anc_s = scramble(anc)
    bd_s = scramble(bd)

    pre_nms = min(_NMS_PRE, scores_flat.shape[1])
    sorted_scores, sort_order = lax.top_k(scores_flat, pre_nms)

    anc_sel = anc_s[0][sort_order][0]
    bd_sel = bd_s[0][sort_order][0]

    def to_tiles(a):
        a = jnp.pad(a, ((0, _NPAD - pre_nms), (0, 0)))
        return a.T.reshape(4, _ROWS, _LANES)

    mask_t, boxes_t = pl.pallas_call(
        _nms_kernel,
        out_shape=(
            jax.ShapeDtypeStruct((_ROWS, _LANES), jnp.float32),
            jax.ShapeDtypeStruct((4, _ROWS, _LANES), jnp.float32),
        ),
    )(to_tiles(anc_sel), to_tiles(bd_sel))

    mask = mask_t.reshape(-1)[:pre_nms] > 0.5
    regs = boxes_t.reshape(4, -1).T[:pre_nms]

    keep = jnp.argsort((~mask).astype(jnp.int32), stable=True)[:_NMS_POST]
    proposals = regs[keep]
    kept_scores = sorted_scores[0][keep]

    out = jnp.zeros((batch_size, _NMS_POST, 5), dtype=cls_scores.dtype)
    out = out.at[0, :, 0].set(kept_scores)
    out = out.at[0, :, 1:].set(proposals)
    return out


# speedup vs baseline: 19.2715x; 1.8190x over previous
import jax
import jax.numpy as jnp
from jax import lax
from jax.experimental import pallas as pl

_RATIOS = (0.5, 1.0, 2.0)
_SCALES = (128.0, 256.0, 512.0)
_IMAGE_SIZE = 1920
_NMS_PRE = 3000
_NMS_POST = 300
_THRESHOLD = 0.6

_NPAD = 3072
_ROWS = 24
_LANES = 128


def _anchors(feat_stride, map_size):
    xs = (feat_stride * jnp.arange(map_size, dtype=jnp.float64) + feat_stride) / 2.0
    ys = xs
    combos = [(r, s) for r in _RATIOS for s in _SCALES]
    r = jnp.array([c[0] for c in combos], dtype=jnp.float64)
    s = jnp.array([c[1] for c in combos], dtype=jnp.float64)
    X = xs[:, None, None]
    Y = ys[None, :, None]
    x1 = X - s / 2.0
    x2 = X + s / 2.0
    y1 = Y - s * r / 2.0
    y2 = Y + s * r / 2.0
    x1, y1, x2, y2 = jnp.broadcast_arrays(x1, y1, x2, y2)
    return jnp.stack([x1, y1, x2, y2], axis=-1)


def _nms_kernel(anc_ref, dlt_ref, mask_ref, boxes_ref):
    x1 = jnp.clip(anc_ref[0] + dlt_ref[0], 0.0, float(_IMAGE_SIZE))
    y1 = jnp.clip(anc_ref[1] + dlt_ref[1], 0.0, float(_IMAGE_SIZE))
    x2 = jnp.clip(anc_ref[2] + dlt_ref[2], 0.0, float(_IMAGE_SIZE))
    y2 = jnp.clip(anc_ref[3] + dlt_ref[3], 0.0, float(_IMAGE_SIZE))
    boxes_ref[0] = x1
    boxes_ref[1] = y1
    boxes_ref[2] = x2
    boxes_ref[3] = y2

    area = (x2 - x1) * (y2 - y1)
    gidx = (
        lax.broadcasted_iota(jnp.int32, (_ROWS, _LANES), 0) * _LANES
        + lax.broadcasted_iota(jnp.int32, (_ROWS, _LANES), 1)
    )

    def step(i, mask):
        sel = gidx == i

        def pick(v):
            return jnp.sum(jnp.where(sel, v, 0.0))

        xi1 = pick(x1)
        yi1 = pick(y1)
        xi2 = pick(x2)
        yi2 = pick(y2)
        mi = jnp.sum(jnp.where(sel, mask, 0.0))
        ai = (xi2 - xi1) * (yi2 - yi1)

        xx1 = jnp.maximum(xi1, x1)
        yy1 = jnp.maximum(yi1, y1)
        xx2 = jnp.minimum(xi2, x2)
        yy2 = jnp.minimum(yi2, y2)
        w = jnp.clip(xx2 - xx1, 0.0, None)
        h = jnp.clip(yy2 - yy1, 0.0, None)
        inter = w * h
        iou = inter / (ai + area - inter + 1e-9)
        sup = (iou > _THRESHOLD) & (gidx > i) & (mi > 0.5)
        return jnp.where(sup, 0.0, mask)

    mask = lax.fori_loop(0, _NMS_PRE, step, jnp.ones((_ROWS, _LANES), jnp.float32))
    mask_ref[...] = mask


def kernel(cls_scores, bbox_deltas):
    H = cls_scores.shape[2]
    W = cls_scores.shape[3]
    batch_size = cls_scores.shape[0]
    feat_stride = round(_IMAGE_SIZE / float(W))
    anc = _anchors(feat_stride, W).astype(jnp.float32)

    scores_flat = cls_scores[0, ::2].reshape(batch_size, -1)

    bd = jnp.transpose(bbox_deltas, (0, 2, 3, 1))[0]
    bd = bd.reshape(H, W, -1, 4)

    def scramble(a):
        a = a.reshape(batch_size, -1, 4, H, W)
        a = jnp.transpose(a, (0, 3, 4, 1, 2)).reshape(batch_size, -1, 4)
        return a

    anc_s = scramble(anc)
    bd_s = scramble(bd)

    pre_nms = min(_NMS_PRE, scores_flat.shape[1])
    neg = jnp.where(scores_flat[0] == 0.0, 0.0, -scores_flat[0])
    sort_order = jnp.argsort(neg, stable=True)[:pre_nms][None]
    sorted_scores = scores_flat[0][sort_order]

    anc_sel = anc_s[0][sort_order][0]
    bd_sel = bd_s[0][sort_order][0]

    def to_tiles(a):
        a = jnp.pad(a, ((0, _NPAD - pre_nms), (0, 0)))
        return a.T.reshape(4, _ROWS, _LANES)

    mask_t, boxes_t = pl.pallas_call(
        _nms_kernel,
        out_shape=(
            jax.ShapeDtypeStruct((_ROWS, _LANES), jnp.float32),
            jax.ShapeDtypeStruct((4, _ROWS, _LANES), jnp.float32),
        ),
    )(to_tiles(anc_sel), to_tiles(bd_sel))

    mask = mask_t.reshape(-1)[:pre_nms] > 0.5
    regs = boxes_t.reshape(4, -1).T[:pre_nms]

    keep = jnp.argsort((~mask).astype(jnp.int32), stable=True)[:_NMS_POST]
    proposals = regs[keep]
    kept_scores = sorted_scores[0][keep]

    out = jnp.zeros((batch_size, _NMS_POST, 5), dtype=cls_scores.dtype)
    out = out.at[0, :, 0].set(kept_scores)
    out = out.at[0, :, 1:].set(proposals)
    return out
